# baseline (speedup 1.0000x reference)
import functools

import jax
import jax.numpy as jnp
from jax import lax
from jax.experimental import pallas as pl
from jax.experimental.pallas import tpu as pltpu
from jax.experimental.pallas import tpu_sc as plsc

NUM_CORES = 2
NUM_SUBCORES = 16
NUM_WORKERS = NUM_CORES * NUM_SUBCORES

NBUF = 8
KAHEAD = 4


def _emb_kernel(B0, S, V, D):
    p_per_w = B0 // NUM_WORKERS
    mesh = plsc.VectorSubcoreMesh(core_axis_name="c", subcore_axis_name="s")

    @functools.partial(
        pl.kernel,
        out_type=jax.ShapeDtypeStruct((B0, S, D), jnp.float32),
        mesh=mesh,
        scratch_types=[
            pltpu.VMEM((p_per_w, S), jnp.int32),
            pltpu.VMEM((NBUF, S, D), jnp.float32),
        ] + [pltpu.SemaphoreType.DMA] * (2 * NBUF),
        compiler_params=pltpu.CompilerParams(use_tc_tiling_on_sc=False),
    )
    def emb(tok_hbm, table_hbm, out_hbm, idx_v, rows_v, *sems):
        gsems, osems = sems[:NBUF], sems[NBUF:]
        wid = lax.axis_index("s") * NUM_CORES + lax.axis_index("c")
        base = wid * p_per_w
        pltpu.sync_copy(tok_hbm.at[pl.ds(base, p_per_w)], idx_v)

        def fire(g, b):
            pltpu.async_copy(table_hbm.at[idx_v.at[g]], rows_v.at[b], gsems[b])

        def drain_gather(b):
            pltpu.make_async_copy(
                table_hbm.at[idx_v.at[0]], rows_v.at[b], gsems[b]).wait()

        def start_out(g, b):
            pltpu.async_copy(rows_v.at[b], out_hbm.at[base + g], osems[b])

        def wait_out(b):
            pltpu.make_async_copy(
                rows_v.at[b], out_hbm.at[base], osems[b]).wait()

        def visit(g, b, bk, do_fire, do_owait):
            if do_fire:
                if do_owait:
                    wait_out(bk)
                fire(g + KAHEAD, bk)
            drain_gather(b)
            start_out(g, b)

        for g in range(KAHEAD):
            fire(g, g % NBUF)
        for g in range(NBUF - KAHEAD):
            visit(g, g % NBUF, (g + KAHEAD) % NBUF, True, False)
        lo, hi = NBUF - KAHEAD, p_per_w - KAHEAD
        assert (hi - lo) % NBUF == 0

        @pl.loop(lo, hi, step=NBUF)
        def _steady(t):
            for i in range(NBUF):
                b = (lo + i) % NBUF
                visit(t + i, b, (b + KAHEAD) % NBUF, True, True)

        for g in range(p_per_w - KAHEAD, p_per_w):
            visit(g, g % NBUF, 0, False, False)
        for b in range(NBUF):
            wait_out(b)

    return emb


def kernel(token_ids, ME):
    B0, S = token_ids.shape
    V, D = ME.shape
    return _emb_kernel(B0, S, V, D)(token_ids, ME)


# speedup vs baseline: 1.0005x; 1.0005x over previous
import functools

import jax
import jax.numpy as jnp
from jax import lax
from jax.experimental import pallas as pl
from jax.experimental.pallas import tpu as pltpu
from jax.experimental.pallas import tpu_sc as plsc

NUM_CORES = 2
NUM_SUBCORES = 16
NUM_WORKERS = NUM_CORES * NUM_SUBCORES

NBUF = 8
KAHEAD = 4


def _emb_kernel(B0, S, V, D):
    p_per_w = B0 // NUM_WORKERS
    mesh = plsc.VectorSubcoreMesh(core_axis_name="c", subcore_axis_name="s")

    @functools.partial(
        pl.kernel,
        out_type=jax.ShapeDtypeStruct((B0, S, D), jnp.float32),
        mesh=mesh,
        scratch_types=[
            pltpu.VMEM((p_per_w, S), jnp.int32),
            pltpu.VMEM((NBUF, S, D), jnp.float32),
        ] + [pltpu.SemaphoreType.DMA] * (2 * NBUF),
        compiler_params=pltpu.CompilerParams(use_tc_tiling_on_sc=False),
    )
    def emb(tok_hbm, table_hbm, out_hbm, idx_v, rows_v, *sems):
        gsems, osems = sems[:NBUF], sems[NBUF:]
        wid = lax.axis_index("s") * NUM_CORES + lax.axis_index("c")
        base = wid * p_per_w
        pltpu.sync_copy(tok_hbm.at[pl.ds(base, p_per_w)], idx_v)

        def fire(g, b):
            pltpu.async_copy(table_hbm.at[idx_v.at[g]], rows_v.at[b], gsems[b])

        def drain_gather(b):
            pltpu.make_async_copy(
                table_hbm.at[idx_v.at[0]], rows_v.at[b], gsems[b]).wait()

        def start_out(g, b):
            pltpu.async_copy(rows_v.at[b], out_hbm.at[base + g], osems[b])

        def wait_out(b):
            pltpu.make_async_copy(
                rows_v.at[b], out_hbm.at[base], osems[b]).wait()

        def visit(g, b, bk, do_fire, do_owait):
            if do_fire:
                if do_owait:
                    wait_out(bk)
                fire(g + KAHEAD, bk)
            drain_gather(b)
            start_out(g, b)

        for g in range(KAHEAD):
            fire(g, g % NBUF)
        for g in range(NBUF - KAHEAD):
            visit(g, g % NBUF, (g + KAHEAD) % NBUF, True, False)
        lo, hi = NBUF - KAHEAD, p_per_w - KAHEAD
        assert (hi - lo) % NBUF == 0

        @pl.loop(lo, hi, step=NBUF)
        def _steady(t):
            for i in range(NBUF):
                b = (lo + i) % NBUF
                visit(t + i, b, (b + KAHEAD) % NBUF, True, True)

        for g in range(p_per_w - KAHEAD, p_per_w):
            visit(g, g % NBUF, 0, False, False)
        for b in range(NBUF):
            wait_out(b)

    return emb


def kernel(token_ids, ME):
    from jax.experimental.layout import Format, Layout
    B0, S = token_ids.shape
    V, D = ME.shape
    sharding = jax.sharding.SingleDeviceSharding(jax.devices()[0])
    MEf = jax.device_put(ME, Format(Layout(major_to_minor=(0, 1),
                                           tiling=((16,),)), sharding))
    return _emb_kernel(B0, S, V, D)(token_ids, MEf)


# speedup vs baseline: 1.0041x; 1.0036x over previous
import functools

import jax
import jax.numpy as jnp
from jax import lax
from jax.experimental import pallas as pl
from jax.experimental.pallas import tpu as pltpu
from jax.experimental.pallas import tpu_sc as plsc

NUM_CORES = 2
NUM_SUBCORES = 16
NUM_WORKERS = NUM_CORES * NUM_SUBCORES

NBUF = 8
KAHEAD = 6


def _emb_kernel(B0, S, V, D):
    p_per_w = B0 // NUM_WORKERS
    mesh = plsc.VectorSubcoreMesh(core_axis_name="c", subcore_axis_name="s")

    @functools.partial(
        pl.kernel,
        out_type=jax.ShapeDtypeStruct((B0, S, D), jnp.float32),
        mesh=mesh,
        scratch_types=[
            pltpu.VMEM((p_per_w, S), jnp.int32),
            pltpu.VMEM((NBUF, S, D), jnp.float32),
        ] + [pltpu.SemaphoreType.DMA] * (2 * NBUF),
        compiler_params=pltpu.CompilerParams(use_tc_tiling_on_sc=False),
    )
    def emb(tok_hbm, table_hbm, out_hbm, idx_v, rows_v, *sems):
        gsems, osems = sems[:NBUF], sems[NBUF:]
        wid = lax.axis_index("s") * NUM_CORES + lax.axis_index("c")
        base = wid * p_per_w
        pltpu.sync_copy(tok_hbm.at[pl.ds(base, p_per_w)], idx_v)

        def fire(g, b):
            pltpu.async_copy(table_hbm.at[idx_v.at[g]], rows_v.at[b], gsems[b])

        def drain_gather(b):
            pltpu.make_async_copy(
                table_hbm.at[idx_v.at[0]], rows_v.at[b], gsems[b]).wait()

        def start_out(g, b):
            pltpu.async_copy(rows_v.at[b], out_hbm.at[base + g], osems[b])

        def wait_out(b):
            pltpu.make_async_copy(
                rows_v.at[b], out_hbm.at[base], osems[b]).wait()

        def visit(g, b, bk, do_fire, do_owait):
            if do_fire:
                if do_owait:
                    wait_out(bk)
                fire(g + KAHEAD, bk)
            drain_gather(b)
            start_out(g, b)

        for g in range(KAHEAD):
            fire(g, g % NBUF)
        for g in range(NBUF - KAHEAD):
            visit(g, g % NBUF, (g + KAHEAD) % NBUF, True, False)
        lo, hi = NBUF - KAHEAD, p_per_w - KAHEAD
        assert (hi - lo) % NBUF == 0

        @pl.loop(lo, hi, step=NBUF)
        def _steady(t):
            for i in range(NBUF):
                b = (lo + i) % NBUF
                visit(t + i, b, (b + KAHEAD) % NBUF, True, True)

        for g in range(p_per_w - KAHEAD, p_per_w):
            visit(g, g % NBUF, 0, False, False)
        for b in range(NBUF):
            wait_out(b)

    return emb


def kernel(token_ids, ME):
    B0, S = token_ids.shape
    V, D = ME.shape
    return _emb_kernel(B0, S, V, D)(token_ids, ME)
